# baseline (speedup 1.0000x reference)
import jax
import jax.numpy as jnp
from jax.experimental import pallas as pl
from jax.experimental.pallas import tpu as pltpu

_BM = 200


def _sanbet_kernel(w_ref, b_ref, adj0_ref, adj1_ref, inp_ref, out_ref):
    x = inp_ref[...].astype(jnp.bfloat16)
    dn = (((1,), (0,)), ((), ()))
    w = w_ref[0, 0]
    b = b_ref[0, 0]
    a0 = adj0_ref[0].astype(jnp.bfloat16)
    acc0 = jax.lax.dot_general(a0, x, dn, preferred_element_type=jnp.float32)
    out_ref[0] = acc0 * w + b
    a1 = adj1_ref[0].astype(jnp.bfloat16)
    acc1 = jax.lax.dot_general(a1, x, dn, preferred_element_type=jnp.float32)
    out_ref[1] = acc1 * w + b


def kernel(inp, adj, weight, bias):
    n, d = inp.shape
    h = n // 2
    w2 = weight.reshape(1, 1)
    b2 = bias.reshape(1, 1)
    adj3 = adj.reshape(2, h, n)
    grid = (h // _BM,)
    out3 = pl.pallas_call(
        _sanbet_kernel,
        grid=grid,
        in_specs=[
            pl.BlockSpec((1, 1), lambda i: (0, 0)),
            pl.BlockSpec((1, 1), lambda i: (0, 0)),
            pl.BlockSpec((1, _BM, n), lambda i: (0, i, 0)),
            pl.BlockSpec((1, _BM, n), lambda i: (1, i, 0)),
            pl.BlockSpec((n, d), lambda i: (0, 0)),
        ],
        out_specs=pl.BlockSpec((2, _BM, d), lambda i: (0, i, 0)),
        out_shape=jax.ShapeDtypeStruct((2, h, d), jnp.float32),
        compiler_params=pltpu.CompilerParams(
            dimension_semantics=("arbitrary",),
        ),
    )(w2, b2, adj3, adj3, inp)
    return out3.reshape(n, d)
